# Initial kernel scaffold; baseline (speedup 1.0000x reference)
import jax
import jax.numpy as jnp
from jax.experimental import pallas as pl


def kernel(x, Wg0, bg0, Wt0, bt0, Wr0, Wg1, bg1, Wt1, bt1, Wg2, bg2, Wt2, bt2):
    raise NotImplementedError("write your pallas kernel here")


# speedup vs baseline: 19.1300x; 19.1300x over previous
import numpy as np
import jax
import jax.numpy as jnp
from jax.experimental import pallas as pl
from jax.experimental.pallas import tpu as pltpu

V = 21
K = 3
T = 256
HALO = 4
TP = T + 2 * HALO
H = 64
_EDGES = [(0, 1), (1, 2), (2, 3), (3, 4), (0, 5), (5, 6), (6, 7), (7, 8),
          (0, 9), (9, 10), (10, 11), (11, 12), (0, 13), (13, 14), (14, 15),
          (15, 16), (0, 17), (17, 18), (18, 19), (19, 20)]


def _adjacency_triples():
    A = np.zeros((V, V), dtype=np.float64)
    for i, j in _EDGES:
        A[i, j] = 1.0
        A[j, i] = 1.0
    d = np.full(V, -1, dtype=np.int64)
    d[0] = 0
    frontier = [0]
    while frontier:
        nxt = []
        for u in frontier:
            for w in range(V):
                if A[u, w] > 0 and d[w] < 0:
                    d[w] = d[u] + 1
                    nxt.append(w)
        frontier = nxt
    Ahat = A + np.eye(V)
    AD = (Ahat / Ahat.sum(axis=0, keepdims=True)).astype(np.float32)
    A0 = AD * (d[:, None] == d[None, :])
    A1 = AD * (d[None, :] < d[:, None])
    A2 = AD * (d[None, :] > d[:, None])
    Astack = np.stack([A0, A1, A2], axis=0)
    by_w = []
    for w in range(V):
        terms = []
        for k in range(K):
            for v in range(V):
                c = float(Astack[k, v, w])
                if c != 0.0:
                    terms.append((k, v, c))
        by_w.append(terms)
    return by_w


_BY_W = _adjacency_triples()


def _stgcn_body(x_ref, wg0, bg0, wt0, bt0, wr0, wg1, bg1, wt1, bt1,
                wg2, bg2, wt2, bt2, out_ref):
    xb = x_ref[0]
    rows = V * TP
    rowtp = jax.lax.broadcasted_iota(jnp.int32, (rows, 1), 0) % TP
    in_range = jnp.logical_and(rowtp >= HALO, rowtp < HALO + T)
    validf = in_range.astype(jnp.float32)

    nzc = jnp.all(xb != 0.0, axis=1, keepdims=True)
    frame_ok = nzc[0:TP]
    for v in range(1, V):
        frame_ok = jnp.logical_and(frame_ok, nzc[v * TP:(v + 1) * TP])
    lens = jnp.sum(frame_ok.astype(jnp.int32))

    def block(h, wg, bg, wt, bt, res):
        gs = []
        for k in range(K):
            g = jnp.dot(h, wg[k], preferred_element_type=jnp.float32)
            gs.append((g + bg[k][None, :]) * validf)
        out_rows = []
        for w in range(V):
            acc = None
            for (k, v, c) in _BY_W[w]:
                term = c * gs[k][v * TP:(v + 1) * TP, :]
                acc = term if acc is None else acc + term
            out_rows.append(acc)
        a = jnp.concatenate(out_rows, axis=0)
        n = rows - 2 * HALO
        t = jnp.dot(a[0:n, :], wt[0], preferred_element_type=jnp.float32)
        for dt in range(1, 9):
            t = t + jnp.dot(a[dt:n + dt, :], wt[dt],
                            preferred_element_type=jnp.float32)
        t = jnp.concatenate(
            [jnp.zeros((HALO, H), jnp.float32), t,
             jnp.zeros((HALO, H), jnp.float32)], axis=0)
        t = (t + bt[0][None, :]) * validf
        return jnp.maximum(t + res, 0.0)

    res0 = jnp.dot(xb, wr0[:], preferred_element_type=jnp.float32)
    h = block(xb, wg0, bg0, wt0, bt0, res0)
    h = block(h, wg1, bg1, wt1, bt1, h)
    h = block(h, wg2, bg2, wt2, bt2, h)

    mask = jnp.logical_and(in_range, (rowtp - HALO) < lens)
    h = h * mask.astype(jnp.float32)
    for v in range(V):
        out_ref[0, v] = h[v * TP + HALO:v * TP + HALO + T, :]


def _prep_gcn(Wg, bg, cin):
    wg = Wg.reshape(cin, K, H).transpose(1, 0, 2)
    return wg, bg.reshape(K, H)


def _prep_tcn(Wt, bt):
    return Wt.transpose(2, 1, 0), bt.reshape(1, H)


def kernel(x, Wg0, bg0, Wt0, bt0, Wr0, Wg1, bg1, Wt1, bt1,
           Wg2, bg2, Wt2, bt2):
    B, Tn, Vn, C = x.shape
    xt = jnp.transpose(x, (0, 2, 1, 3))
    xp = jnp.pad(xt, ((0, 0), (0, 0), (HALO, HALO), (0, 0)))
    xp = xp.reshape(B, V * TP, C)

    wg0, bg0r = _prep_gcn(Wg0, bg0, C)
    wg1, bg1r = _prep_gcn(Wg1, bg1, H)
    wg2, bg2r = _prep_gcn(Wg2, bg2, H)
    wt0, bt0r = _prep_tcn(Wt0, bt0)
    wt1, bt1r = _prep_tcn(Wt1, bt1)
    wt2, bt2r = _prep_tcn(Wt2, bt2)

    full = lambda shape: pl.BlockSpec(shape, lambda b: (0,) * len(shape))
    out = pl.pallas_call(
        _stgcn_body,
        grid=(B,),
        in_specs=[
            pl.BlockSpec((1, V * TP, C), lambda b: (b, 0, 0)),
            full((K, C, H)), full((K, H)), full((9, H, H)), full((1, H)),
            full((C, H)),
            full((K, H, H)), full((K, H)), full((9, H, H)), full((1, H)),
            full((K, H, H)), full((K, H)), full((9, H, H)), full((1, H)),
        ],
        out_specs=pl.BlockSpec((1, V, T, H), lambda b: (b, 0, 0, 0)),
        out_shape=jax.ShapeDtypeStruct((B, V, T, H), jnp.float32),
        compiler_params=pltpu.CompilerParams(
            dimension_semantics=("parallel",)),
    )(xp, wg0, bg0r, wt0, bt0r, Wr0, wg1, bg1r, wt1, bt1r,
      wg2, bg2r, wt2, bt2r)
    return jnp.transpose(out, (0, 3, 2, 1))
